# baseline (speedup 1.0000x reference)
from functools import partial

import jax
import jax.numpy as jnp
from jax.experimental import pallas as pl
from jax.experimental.pallas import tpu as pltpu


def _gather_kernel(idx_ref, vloc_ref, wloc_ref, du0_ref, du1_ref, tab_ref,
                   out_ref, gtile, tab3, idx_smem, sem, *, TMQ, ROWS_I):
    t = pl.program_id(0)
    nt = pl.num_programs(0)
    slot = jax.lax.rem(t, 2)
    nxt_slot = jax.lax.rem(t + 1, 2)
    nxt = jnp.minimum(t + 1, nt - 1)

    @pl.when(t == 0)
    def _prologue():
        pltpu.make_async_copy(idx_ref.at[0], idx_smem.at[0], sem.at[0]).start()
        cpt = pltpu.make_async_copy(tab_ref, tab3.at[:, 0], sem.at[2])
        cpt.start()
        cpt.wait()

    pltpu.make_async_copy(idx_ref.at[nxt], idx_smem.at[nxt_slot],
                          sem.at[nxt_slot]).start()
    pltpu.make_async_copy(idx_ref.at[t], idx_smem.at[slot],
                          sem.at[slot]).wait()

    lane = jax.lax.broadcasted_iota(jnp.int32, (TMQ, 128), 1)
    djm = (lane >> 6).astype(jnp.float32)
    zzm = (lane & 7).astype(jnp.float32)
    wv = jnp.maximum(0.0, 1.0 - jnp.abs(vloc_ref[...] - djm))
    wz = jnp.maximum(0.0, 1.0 - jnp.abs(wloc_ref[...] - zzm))
    wvz = wv * wz
    wu0 = jnp.maximum(0.0, 1.0 - du0_ref[...])
    wu1 = jnp.maximum(0.0, 1.0 - du1_ref[...])

    sl = jax.lax.broadcasted_iota(jnp.int32, (128, 8), 0)
    sc = jax.lax.broadcasted_iota(jnp.int32, (128, 8), 1)
    smat = (((sl >> 3) & 7) == sc).astype(jnp.float32)

    for q in range(TMQ):
        b = idx_smem[slot, q]
        gtile[q, :] = tab3[b, 0]
        gtile[TMQ + q, :] = tab3[b + ROWS_I, 0]

    p0 = gtile[0:TMQ, :] * (wvz * wu0)
    p1 = gtile[TMQ:, :] * (wvz * wu1)
    dn = (((0,), (1,)), ((), ()))
    o0 = jax.lax.dot_general(smat, p0, dn,
                             preferred_element_type=jnp.float32)
    o1 = jax.lax.dot_general(smat, p1, dn,
                             preferred_element_type=jnp.float32)
    out_ref[...] = o0 + o1

    @pl.when(t == nt - 1)
    def _epilogue():
        pltpu.make_async_copy(idx_ref.at[nxt], idx_smem.at[nxt_slot],
                              sem.at[nxt_slot]).wait()


def _table_builder_kernel(g_ref, out_ref, *, Ny, nzb):
    for kb in range(nzb):
        pieces = []
        for dj in range(2):
            for c in range(8):
                r0 = c * (Ny + 1) + dj
                pieces.append(g_ref[r0:r0 + Ny, 4 * kb:4 * kb + 8])
        out_ref[kb, :, :] = jnp.concatenate(pieces, axis=1)


def _build_table(grid):
    _, C, Nx, Ny, Nz = grid.shape
    nzb = Nz // 4
    g = jnp.pad(grid[0].astype(jnp.float32),
                ((0, 0), (0, 1), (0, 1), (0, 128 - Nz)))
    gi = g.transpose(1, 0, 2, 3).reshape((Nx + 1) * C * (Ny + 1), 128)
    tab = pl.pallas_call(
        partial(_table_builder_kernel, Ny=Ny, nzb=nzb),
        out_shape=jax.ShapeDtypeStruct((nzb, (Nx + 1) * Ny, 128),
                                       jnp.float32),
        grid=(Nx + 1,),
        in_specs=[pl.BlockSpec((C * (Ny + 1), 128), lambda i: (i, 0))],
        out_specs=pl.BlockSpec((nzb, Ny, 128), lambda i: (0, i, 0)),
        compiler_params=pltpu.CompilerParams(
            dimension_semantics=("parallel",),
        ),
    )(gi)
    return tab.reshape(nzb * (Nx + 1) * Ny, 128)


def kernel(query, grid, xyz_min, xyz_max):
    _, C, Nx, Ny, Nz = grid.shape
    assert C == 8 and Nz % 4 == 0
    lead_shape = query.shape[:-1]

    q = query.reshape(-1, 3).astype(jnp.float32)
    M = q.shape[0]
    t = (q - xyz_min) / (xyz_max - xyz_min)
    u = t[:, 0] * (Nx - 1)
    v = t[:, 1] * (Ny - 1)
    w = t[:, 2] * (Nz - 1)

    cif = jnp.clip(jnp.floor(u), 0.0, Nx - 1.0)
    cjf = jnp.clip(jnp.floor(v), 0.0, Ny - 1.0)
    ckf = jnp.clip(jnp.floor(w), 0.0, Nz - 1.0)
    kb = ckf.astype(jnp.int32) >> 2
    nzb = Nz // 4
    idx = (kb * ((Nx + 1) * Ny)
           + cif.astype(jnp.int32) * Ny + cjf.astype(jnp.int32))
    vloc = v - cjf
    wloc = w - 4.0 * kb.astype(jnp.float32)
    du0 = jnp.abs(u - cif)
    du1 = jnp.abs(u - (cif + 1.0))

    TMQ = 2048
    M_pad = pl.cdiv(M, TMQ) * TMQ
    pad = M_pad - M
    idx = jnp.pad(idx, (0, pad))
    vloc = jnp.pad(vloc, (0, pad))
    wloc = jnp.pad(wloc, (0, pad))
    du0 = jnp.pad(du0, (0, pad))
    du1 = jnp.pad(du1, (0, pad))
    NT = M_pad // TMQ

    idx3 = idx.reshape(NT, TMQ)
    vloc = vloc.reshape(M_pad, 1)
    wloc = wloc.reshape(M_pad, 1)
    du0 = du0.reshape(M_pad, 1)
    du1 = du1.reshape(M_pad, 1)

    tab = _build_table(grid)
    ROWS_I = Ny

    out = pl.pallas_call(
        partial(_gather_kernel, TMQ=TMQ, ROWS_I=ROWS_I),
        out_shape=jax.ShapeDtypeStruct((8, M_pad), jnp.float32),
        grid=(NT,),
        in_specs=[
            pl.BlockSpec((NT, TMQ), lambda m: (0, 0)),
            pl.BlockSpec((TMQ, 1), lambda m: (m, 0)),
            pl.BlockSpec((TMQ, 1), lambda m: (m, 0)),
            pl.BlockSpec((TMQ, 1), lambda m: (m, 0)),
            pl.BlockSpec((TMQ, 1), lambda m: (m, 0)),
            pl.BlockSpec(memory_space=pl.ANY),
        ],
        out_specs=pl.BlockSpec((8, TMQ), lambda m: (0, m)),
        scratch_shapes=[
            pltpu.VMEM((2 * TMQ, 128), jnp.float32),
            pltpu.VMEM((tab.shape[0], 1, 128), jnp.float32),
            pltpu.SMEM((2, TMQ), jnp.int32),
            pltpu.SemaphoreType.DMA((3,)),
        ],
        compiler_params=pltpu.CompilerParams(
            dimension_semantics=("arbitrary",),
            vmem_limit_bytes=56 * 1024 * 1024,
        ),
    )(idx3, vloc, wloc, du0, du1, tab)

    return out[:, :M].T.reshape(*lead_shape, C)


# speedup vs baseline: 2.3225x; 2.3225x over previous
from functools import partial

import jax
import jax.numpy as jnp
from jax.experimental import pallas as pl
from jax.experimental.pallas import tpu as pltpu


def _gather_kernel(idx_ref, vloc_ref, wloc_ref, du0_ref, du1_ref, tab_ref,
                   out_ref, gtile, tab3, idx_smem, sem, *, TMQ, ROWS_I):
    t = pl.program_id(0)
    nt = pl.num_programs(0)
    slot = jax.lax.rem(t, 2)
    nxt_slot = jax.lax.rem(t + 1, 2)
    nxt = jnp.minimum(t + 1, nt - 1)

    @pl.when(t == 0)
    def _prologue():
        pltpu.make_async_copy(idx_ref.at[0], idx_smem.at[0], sem.at[0]).start()
        cpt = pltpu.make_async_copy(tab_ref, tab3.at[:, 0], sem.at[2])
        cpt.start()
        cpt.wait()

    pltpu.make_async_copy(idx_ref.at[nxt], idx_smem.at[nxt_slot],
                          sem.at[nxt_slot]).start()
    pltpu.make_async_copy(idx_ref.at[t], idx_smem.at[slot],
                          sem.at[slot]).wait()

    lane = jax.lax.broadcasted_iota(jnp.int32, (TMQ, 128), 1)
    djm = (lane >> 6).astype(jnp.float32)
    zzm = (lane & 7).astype(jnp.float32)
    vcol = jnp.transpose(vloc_ref[...], (1, 0))
    wcol = jnp.transpose(wloc_ref[...], (1, 0))
    wv = jnp.maximum(0.0, 1.0 - jnp.abs(vcol - djm))
    wz = jnp.maximum(0.0, 1.0 - jnp.abs(wcol - zzm))
    wvz = wv * wz
    wu0 = jnp.maximum(0.0, 1.0 - du0_ref[...])
    wu1 = jnp.maximum(0.0, 1.0 - du1_ref[...])

    sl = jax.lax.broadcasted_iota(jnp.int32, (128, 8), 0)
    sc = jax.lax.broadcasted_iota(jnp.int32, (128, 8), 1)
    smat = (((sl >> 3) & 7) == sc).astype(jnp.float32)

    for q in range(TMQ):
        b = idx_smem[slot, q]
        gtile[q, :] = tab3[b, 0]
        gtile[TMQ + q, :] = tab3[b + ROWS_I, 0]

    p0 = gtile[0:TMQ, :] * wvz
    p1 = gtile[TMQ:, :] * wvz
    dn = (((0,), (1,)), ((), ()))
    o0 = jax.lax.dot_general(smat, p0, dn,
                             preferred_element_type=jnp.float32)
    o1 = jax.lax.dot_general(smat, p1, dn,
                             preferred_element_type=jnp.float32)
    out_ref[...] = o0 * wu0 + o1 * wu1

    @pl.when(t == nt - 1)
    def _epilogue():
        pltpu.make_async_copy(idx_ref.at[nxt], idx_smem.at[nxt_slot],
                              sem.at[nxt_slot]).wait()


def _table_builder_kernel(g_ref, out_ref, *, Ny, nzb):
    for kb in range(nzb):
        pieces = []
        for dj in range(2):
            for c in range(8):
                r0 = c * (Ny + 1) + dj
                pieces.append(g_ref[r0:r0 + Ny, 4 * kb:4 * kb + 8])
        out_ref[kb, :, :] = jnp.concatenate(pieces, axis=1)


def _build_table(grid):
    _, C, Nx, Ny, Nz = grid.shape
    nzb = Nz // 4
    g = jnp.pad(grid[0].astype(jnp.float32),
                ((0, 0), (0, 1), (0, 1), (0, 128 - Nz)))
    gi = g.transpose(1, 0, 2, 3).reshape((Nx + 1) * C * (Ny + 1), 128)
    tab = pl.pallas_call(
        partial(_table_builder_kernel, Ny=Ny, nzb=nzb),
        out_shape=jax.ShapeDtypeStruct((nzb, (Nx + 1) * Ny, 128),
                                       jnp.float32),
        grid=(Nx + 1,),
        in_specs=[pl.BlockSpec((C * (Ny + 1), 128), lambda i: (i, 0))],
        out_specs=pl.BlockSpec((nzb, Ny, 128), lambda i: (0, i, 0)),
        compiler_params=pltpu.CompilerParams(
            dimension_semantics=("parallel",),
        ),
    )(gi)
    return tab.reshape(nzb * (Nx + 1) * Ny, 128)


def kernel(query, grid, xyz_min, xyz_max):
    _, C, Nx, Ny, Nz = grid.shape
    assert C == 8 and Nz % 4 == 0
    lead_shape = query.shape[:-1]

    q = query.reshape(-1, 3).astype(jnp.float32)
    M = q.shape[0]
    t = (q - xyz_min) / (xyz_max - xyz_min)
    u = t[:, 0] * (Nx - 1)
    v = t[:, 1] * (Ny - 1)
    w = t[:, 2] * (Nz - 1)

    cif = jnp.clip(jnp.floor(u), 0.0, Nx - 1.0)
    cjf = jnp.clip(jnp.floor(v), 0.0, Ny - 1.0)
    ckf = jnp.clip(jnp.floor(w), 0.0, Nz - 1.0)
    kb = ckf.astype(jnp.int32) >> 2
    nzb = Nz // 4
    idx = (kb * ((Nx + 1) * Ny)
           + cif.astype(jnp.int32) * Ny + cjf.astype(jnp.int32))
    vloc = v - cjf
    wloc = w - 4.0 * kb.astype(jnp.float32)
    du0 = jnp.abs(u - cif)
    du1 = jnp.abs(u - (cif + 1.0))

    TMQ = 2048
    M_pad = pl.cdiv(M, TMQ) * TMQ
    pad = M_pad - M
    idx = jnp.pad(idx, (0, pad))
    vloc = jnp.pad(vloc, (0, pad))
    wloc = jnp.pad(wloc, (0, pad))
    du0 = jnp.pad(du0, (0, pad))
    du1 = jnp.pad(du1, (0, pad))
    NT = M_pad // TMQ

    idx3 = idx.reshape(NT, TMQ)
    vloc = vloc.reshape(1, M_pad)
    wloc = wloc.reshape(1, M_pad)
    du0 = du0.reshape(1, M_pad)
    du1 = du1.reshape(1, M_pad)

    tab = _build_table(grid)
    ROWS_I = Ny

    out = pl.pallas_call(
        partial(_gather_kernel, TMQ=TMQ, ROWS_I=ROWS_I),
        out_shape=jax.ShapeDtypeStruct((8, M_pad), jnp.float32),
        grid=(NT,),
        in_specs=[
            pl.BlockSpec((NT, TMQ), lambda m: (0, 0)),
            pl.BlockSpec((1, TMQ), lambda m: (0, m)),
            pl.BlockSpec((1, TMQ), lambda m: (0, m)),
            pl.BlockSpec((1, TMQ), lambda m: (0, m)),
            pl.BlockSpec((1, TMQ), lambda m: (0, m)),
            pl.BlockSpec(memory_space=pl.ANY),
        ],
        out_specs=pl.BlockSpec((8, TMQ), lambda m: (0, m)),
        scratch_shapes=[
            pltpu.VMEM((2 * TMQ, 128), jnp.float32),
            pltpu.VMEM((tab.shape[0], 1, 128), jnp.float32),
            pltpu.SMEM((2, TMQ), jnp.int32),
            pltpu.SemaphoreType.DMA((3,)),
        ],
        compiler_params=pltpu.CompilerParams(
            dimension_semantics=("arbitrary",),
            vmem_limit_bytes=56 * 1024 * 1024,
        ),
    )(idx3, vloc, wloc, du0, du1, tab)

    return out[:, :M].T.reshape(*lead_shape, C)
